# baseline (speedup 1.0000x reference)
import functools

import jax
import jax.numpy as jnp
from jax import lax
from jax.experimental import pallas as pl
from jax.experimental.pallas import tpu as pltpu
from jax.experimental.pallas import tpu_sc as plsc

B = 4096
L = 50
T = 10
D = 64
NU = 100000
NI1 = 100001

NC = 2
NS = 16
NW = NC * NS
BPT = B // NW
SB = 8
NSTEP = BPT // SB
XCOLS = 100
XG = SB * L // XCOLS
PPS = SB * T
NGRP = PPS // 16

RB = 128
RBLK = RB * D


def _relayout_body(euiT, eiuT, eilT, eliT, tEUI, tEIU, tEIL, tELI,
                   oEUI, oEIU, oEIL, oELI, in_buf, in_buf2, out_buf,
                   out_buf2, isem, isem2, osem, osem2):
    wid = lax.axis_index("s") * NC + lax.axis_index("c")
    iot = lax.iota(jnp.int32, 16)
    rotp = {s: (iot + s) & 15 for s in (8, 4, 2, 1)}
    rotm = {s: (iot - s) & 15 for s in (8, 4, 2, 1)}
    masks = {s: (iot & s) != 0 for s in (8, 4, 2, 1)}

    for ti, (tbl, tail, out, n_rows) in enumerate(
            ((euiT, tEUI, oEUI, NU), (eiuT, tEIU, oEIU, NI1),
             (eilT, tEIL, oEIL, NI1), (eliT, tELI, oELI, NI1))):
        nfull = n_rows // RB
        nt = n_rows - nfull * RB

        @pl.when(wid == NW - 1 - ti)
        def _tail(tail=tail, out=out, nfull=nfull, nt=nt):
            pltpu.sync_copy(tail, out_buf.at[pl.ds(0, nt * D)])
            pltpu.sync_copy(out_buf.at[pl.ds(0, nt * D)],
                            out.at[pl.ds(nfull * RB * D, nt * D)])

        def fire_in(bid, buf, sem, tbl=tbl):
            r0 = pl.multiple_of(bid * RB, RB)
            for dhi in range(D // 8):
                pltpu.async_copy(tbl.at[pl.ds(dhi * 8, 8), pl.ds(r0, RB)],
                                 buf.at[pl.ds(dhi * 8, 8)], sem)

        def wait_in(bid, buf, sem, tbl=tbl):
            r0 = pl.multiple_of(bid * RB, RB)
            for dhi in range(D // 8):
                pltpu.make_async_copy(
                    tbl.at[pl.ds(dhi * 8, 8), pl.ds(r0, RB)],
                    buf.at[pl.ds(dhi * 8, 8)], sem).wait()

        def process(bid, buf, obuf, sem, out=out):
            r0 = pl.multiple_of(bid * RB, RB)

            @pl.when(bid >= 2 * NW)
            def _drain():
                pltpu.make_async_copy(
                    obuf, out.at[pl.ds(r0 * D, RBLK)], sem).wait()

            def rloop(rh, _2):
                for d0 in range(0, D, 16):
                    v = [buf[d0 + i, pl.ds(rh * 16, 16)] for i in range(16)]
                    for s in (8, 4, 2, 1):
                        m = masks[s]
                        for i in range(16):
                            if i & s:
                                continue
                            a, b = v[i], v[i | s]
                            ra = a.at[rotp[s]].get(mode="promise_in_bounds")
                            rb = b.at[rotm[s]].get(mode="promise_in_bounds")
                            v[i] = jnp.where(m, rb, a)
                            v[i | s] = jnp.where(m, b, ra)
                    for j in range(16):
                        obuf[pl.ds(rh * 16 * D + j * D + d0, 16)] = v[j]
                return _2

            lax.fori_loop(0, RB // 16, rloop, None)
            pltpu.async_copy(obuf, out.at[pl.ds(r0 * D, RBLK)], sem)

        npair = ((nfull + NW - 1) // NW + 1) // 2

        @pl.when(wid < nfull)
        def _prime():
            fire_in(wid, in_buf, isem)

        def bloop(i, _, nfull=nfull, out=out):
            b0 = wid + (2 * i) * NW
            b1 = wid + (2 * i + 1) * NW
            b2 = wid + (2 * i + 2) * NW

            @pl.when(b1 < nfull)
            def _f1():
                fire_in(b1, in_buf2, isem2)

            @pl.when(b0 < nfull)
            def _p0():
                wait_in(b0, in_buf, isem)
                process(b0, in_buf, out_buf, osem)

            @pl.when(b2 < nfull)
            def _f2():
                fire_in(b2, in_buf, isem)

            @pl.when(b1 < nfull)
            def _p1():
                wait_in(b1, in_buf2, isem2)
                process(b1, in_buf2, out_buf2, osem2)
            return _

        lax.fori_loop(0, npair, bloop, None)

        nb = (nfull - wid + NW - 1) // NW

        @pl.when(nb >= 1)
        def _d0(out=out):
            pltpu.make_async_copy(out_buf, out.at[pl.ds(0, RBLK)],
                                  osem).wait()

        @pl.when(nb >= 2)
        def _d1(out=out):
            pltpu.make_async_copy(out_buf2, out.at[pl.ds(0, RBLK)],
                                  osem2).wait()


@functools.partial(
    pl.kernel,
    out_type=(jax.ShapeDtypeStruct((NU * D,), jnp.float32),
              jax.ShapeDtypeStruct((NI1 * D,), jnp.float32),
              jax.ShapeDtypeStruct((NI1 * D,), jnp.float32),
              jax.ShapeDtypeStruct((NI1 * D,), jnp.float32)),
    mesh=plsc.VectorSubcoreMesh(core_axis_name="c", subcore_axis_name="s"),
    scratch_types=[
        pltpu.VMEM((D, RB), jnp.float32),
        pltpu.VMEM((D, RB), jnp.float32),
        pltpu.VMEM((RBLK,), jnp.float32),
        pltpu.VMEM((RBLK,), jnp.float32),
        pltpu.SemaphoreType.DMA,
        pltpu.SemaphoreType.DMA,
        pltpu.SemaphoreType.DMA,
        pltpu.SemaphoreType.DMA,
    ],
    compiler_params=pltpu.CompilerParams(needs_layout_passes=False,
                                         use_tc_tiling_on_sc=True),
)
def _relayout_sc(euiT, eiuT, eilT, eliT, tEUI, tEIU, tEIL, tELI,
                 oEUI, oEIU, oEIL, oELI, *scratch):
    _relayout_body(euiT, eiuT, eilT, eliT, tEUI, tEIU, tEIL, tELI,
                   oEUI, oEIU, oEIL, oELI, *scratch)


def _sc_body(u_hbm, x_hbm, tar_hbm, neg_hbm, off_hbm, eui_t, eiu_t, eil_t,
             eli_t, outT_hbm, outN_hbm,
             u_v, x_v, tar_v, neg_v, off_v, eui_v,
             eli_rowsA, tarU_rowsA, tarL_rowsA, negU_rowsA, negL_rowsA,
             eli_rowsB, tarU_rowsB, tarL_rowsB, negU_rowsB, negL_rowsB,
             trT_v, trN_v, scT_v, scN_v, gsemA, gsemB, esem):
    wid = lax.axis_index("s") * NC + lax.axis_index("c")
    base = wid * BPT
    bufsA = (eli_rowsA, tarU_rowsA, tarL_rowsA, negU_rowsA, negL_rowsA)
    bufsB = (eli_rowsB, tarU_rowsB, tarL_rowsB, negU_rowsB, negL_rowsB)

    pltpu.sync_copy(u_hbm.at[pl.ds(base, BPT)], u_v)
    pltpu.sync_copy(x_hbm.at[pl.ds(wid * (BPT * L // XCOLS), BPT * L // XCOLS)], x_v)
    pltpu.sync_copy(tar_hbm.at[pl.ds(wid * NSTEP, NSTEP)], tar_v)
    pltpu.sync_copy(neg_hbm.at[pl.ds(wid * NSTEP, NSTEP)], neg_v)
    pltpu.sync_copy(off_hbm.at[pl.ds(base, BPT)], off_v)

    pltpu.async_copy(eui_t.at[u_v], eui_v, esem).wait()

    iot16 = lax.iota(jnp.int32, 16)

    def _copies(j, bufs, sem):
        eli_rows, tarU_rows, tarL_rows, negU_rows, negL_rows = bufs
        cps = []
        for g in range(XG):
            cps.append(pltpu.make_async_copy(
                eli_t.at[x_v.at[j * XG + g]],
                eli_rows.at[pl.ds(g * XCOLS, XCOLS)], sem))
        cps.append(pltpu.make_async_copy(eiu_t.at[tar_v.at[j]], tarU_rows, sem))
        cps.append(pltpu.make_async_copy(eil_t.at[tar_v.at[j]], tarL_rows, sem))
        cps.append(pltpu.make_async_copy(eiu_t.at[neg_v.at[j]], negU_rows, sem))
        cps.append(pltpu.make_async_copy(eil_t.at[neg_v.at[j]], negL_rows, sem))
        return cps

    def fire(j, bufs, sem):
        for c in _copies(j, bufs, sem):
            c.start()

    def drain(j, bufs, sem):
        for c in _copies(j, bufs, sem):
            c.wait()

    def compute(j, bufs):
        eli_rows, tarU_rows, tarL_rows, negU_rows, negL_rows = bufs
        for b in range(SB):
            bg = j * SB + b
            a = [eui_v[bg, pl.ds(16 * k, 16)] for k in range(4)]

            def lbody(l, e, b=b):
                r = b * L + l
                return tuple(e[k] + eli_rows[r, pl.ds(16 * k, 16)]
                             for k in range(4))

            e = lax.fori_loop(
                0, L, lbody,
                tuple(jnp.zeros((16,), jnp.float32) for _ in range(4)),
                unroll=10)
            off_s = plsc.load_gather(
                off_v, [jnp.full((16,), bg, jnp.int32)]) * (1.0 / L)
            e = [ek * off_s for ek in e]

            for t in range(T):
                r = b * T + t
                accT = a[0] * tarU_rows[r, pl.ds(0, 16)]
                accN = a[0] * negU_rows[r, pl.ds(0, 16)]
                for k in range(1, 4):
                    accT = accT + a[k] * tarU_rows[r, pl.ds(16 * k, 16)]
                    accN = accN + a[k] * negU_rows[r, pl.ds(16 * k, 16)]
                for k in range(4):
                    accT = accT + e[k] * tarL_rows[r, pl.ds(16 * k, 16)]
                    accN = accN + e[k] * negL_rows[r, pl.ds(16 * k, 16)]
                trT_v[pl.ds(r * 16, 16)] = accT
                trN_v[pl.ds(r * 16, 16)] = accN

        for g in range(NGRP):
            sT = jnp.zeros((16,), jnp.float32)
            sN = jnp.zeros((16,), jnp.float32)
            for c in range(16):
                idx = iot16 * 16 + (g * 256 + c)
                sT = sT + plsc.load_gather(trT_v, [idx])
                sN = sN + plsc.load_gather(trN_v, [idx])
            scT_v[pl.ds(j * PPS + g * 16, 16)] = sT
            scN_v[pl.ds(j * PPS + g * 16, 16)] = sN

    fire(0, bufsA, gsemA)

    def steppair(i, _):
        j0 = 2 * i
        j1 = 2 * i + 1
        fire(j1, bufsB, gsemB)
        drain(j0, bufsA, gsemA)
        compute(j0, bufsA)

        @pl.when(j1 + 1 < NSTEP)
        def _f():
            fire(j1 + 1, bufsA, gsemA)

        drain(j1, bufsB, gsemB)
        compute(j1, bufsB)
        return _

    lax.fori_loop(0, NSTEP // 2, steppair, None)

    pltpu.sync_copy(scT_v, outT_hbm.at[pl.ds(base * T, BPT * T)])
    pltpu.sync_copy(scN_v, outN_hbm.at[pl.ds(base * T, BPT * T)])


@functools.partial(
    pl.kernel,
    out_type=(jax.ShapeDtypeStruct((B * T,), jnp.float32),
              jax.ShapeDtypeStruct((B * T,), jnp.float32)),
    mesh=plsc.VectorSubcoreMesh(core_axis_name="c", subcore_axis_name="s"),
    scratch_types=[
        pltpu.VMEM((BPT,), jnp.int32),
        pltpu.VMEM((B * L // NW // XCOLS, XCOLS), jnp.int32),
        pltpu.VMEM((NSTEP, PPS), jnp.int32),
        pltpu.VMEM((NSTEP, PPS), jnp.int32),
        pltpu.VMEM((BPT,), jnp.float32),
        pltpu.VMEM((BPT, D), jnp.float32),
        pltpu.VMEM((SB * L, D), jnp.float32),
        pltpu.VMEM((PPS, D), jnp.float32),
        pltpu.VMEM((PPS, D), jnp.float32),
        pltpu.VMEM((PPS, D), jnp.float32),
        pltpu.VMEM((PPS, D), jnp.float32),
        pltpu.VMEM((SB * L, D), jnp.float32),
        pltpu.VMEM((PPS, D), jnp.float32),
        pltpu.VMEM((PPS, D), jnp.float32),
        pltpu.VMEM((PPS, D), jnp.float32),
        pltpu.VMEM((PPS, D), jnp.float32),
        pltpu.VMEM((PPS * 16,), jnp.float32),
        pltpu.VMEM((PPS * 16,), jnp.float32),
        pltpu.VMEM((BPT * T,), jnp.float32),
        pltpu.VMEM((BPT * T,), jnp.float32),
        pltpu.SemaphoreType.DMA,
        pltpu.SemaphoreType.DMA,
        pltpu.SemaphoreType.DMA,
    ],
    compiler_params=pltpu.CompilerParams(needs_layout_passes=False,
                                         use_tc_tiling_on_sc=False),
)
def _fpmc_sc(u, x2, tar2, neg2, off, EUI, EIU, EIL, ELI, outT, outN, *scratch):
    _sc_body(u, x2, tar2, neg2, off, EUI, EIU, EIL, ELI, outT, outN, *scratch)


def kernel(u, x, tar, neg, offset, isEval, EUI, EIU, EIL, ELI):
    nf = (NU // RB) * RB
    tEUI = EUI[nf:].reshape(-1)
    tEIU = EIU[nf:].reshape(-1)
    tEIL = EIL[nf:].reshape(-1)
    tELI = ELI[nf:].reshape(-1)
    eui_f, eiu_f, eil_f, eli_f = _relayout_sc(
        EUI.T, EIU.T, EIL.T, ELI.T, tEUI, tEIU, tEIL, tELI)
    EUIr = eui_f.reshape(NU, D)
    EIUr = eiu_f.reshape(NI1, D)
    EILr = eil_f.reshape(NI1, D)
    ELIr = eli_f.reshape(NI1, D)

    x2 = x.reshape(B * L // XCOLS, XCOLS)
    tar2 = tar.reshape(B * T // PPS, PPS)
    neg2 = neg.reshape(B * T // PPS, PPS)
    off = offset.reshape(B)
    sT, sN = _fpmc_sc(u, x2, tar2, neg2, off, EUIr, EIUr, EILr, ELIr)
    sT = sT.reshape(B, T)
    sN = sN.reshape(B, T)
    second = jnp.where(jnp.asarray(isEval), jnp.zeros_like(sN), sN)
    return (sT, second)


# speedup vs baseline: 1.0620x; 1.0620x over previous
import functools

import jax
import jax.numpy as jnp
from jax import lax
from jax.experimental import pallas as pl
from jax.experimental.pallas import tpu as pltpu
from jax.experimental.pallas import tpu_sc as plsc

B = 4096
L = 50
T = 10
D = 64
NU = 100000
NI1 = 100001

NC = 2
NS = 16
NW = NC * NS
BPT = B // NW
SB = 8
NSTEP = BPT // SB
XCOLS = 100
XG = SB * L // XCOLS
PPS = SB * T
NGRP = PPS // 16

RB = 128
RBLK = RB * D


def _relayout_body(euiT, eiuT, eilT, eliT, tEUI, tEIU, tEIL, tELI,
                   oEUI, oEIU, oEIL, oELI, in_buf, in_buf2, out_buf,
                   out_buf2, isem, isem2, osem, osem2):
    wid = lax.axis_index("s") * NC + lax.axis_index("c")
    iot = lax.iota(jnp.int32, 16)
    rotp = {s: (iot + s) & 15 for s in (8, 4, 2, 1)}
    rotm = {s: (iot - s) & 15 for s in (8, 4, 2, 1)}
    masks = {s: (iot & s) != 0 for s in (8, 4, 2, 1)}

    for ti, (tbl, tail, out, n_rows) in enumerate(
            ((euiT, tEUI, oEUI, NU), (eiuT, tEIU, oEIU, NI1),
             (eilT, tEIL, oEIL, NI1), (eliT, tELI, oELI, NI1))):
        nfull = n_rows // RB
        nt = n_rows - nfull * RB

        @pl.when(wid == NW - 1 - ti)
        def _tail(tail=tail, out=out, nfull=nfull, nt=nt):
            pltpu.sync_copy(tail, out_buf.at[pl.ds(0, nt * D)])
            pltpu.sync_copy(out_buf.at[pl.ds(0, nt * D)],
                            out.at[pl.ds(nfull * RB * D, nt * D)])

        def fire_in(bid, buf, sem, tbl=tbl):
            r0 = pl.multiple_of(bid * RB, RB)
            for dhi in range(D // 8):
                pltpu.async_copy(tbl.at[pl.ds(dhi * 8, 8), pl.ds(r0, RB)],
                                 buf.at[pl.ds(dhi * 8, 8)], sem)

        def wait_in(bid, buf, sem, tbl=tbl):
            r0 = pl.multiple_of(bid * RB, RB)
            for dhi in range(D // 8):
                pltpu.make_async_copy(
                    tbl.at[pl.ds(dhi * 8, 8), pl.ds(r0, RB)],
                    buf.at[pl.ds(dhi * 8, 8)], sem).wait()

        def process(bid, buf, obuf, sem, out=out):
            r0 = pl.multiple_of(bid * RB, RB)

            @pl.when(bid >= 2 * NW)
            def _drain():
                pltpu.make_async_copy(
                    obuf, out.at[pl.ds(r0 * D, RBLK)], sem).wait()

            def rloop(rh, _2):
                for d0 in range(0, D, 16):
                    v = [buf[d0 + i, pl.ds(rh * 16, 16)] for i in range(16)]
                    for s in (8, 4, 2, 1):
                        m = masks[s]
                        for i in range(16):
                            if i & s:
                                continue
                            a, b = v[i], v[i | s]
                            ra = a.at[rotp[s]].get(mode="promise_in_bounds")
                            rb = b.at[rotm[s]].get(mode="promise_in_bounds")
                            v[i] = jnp.where(m, rb, a)
                            v[i | s] = jnp.where(m, b, ra)
                    for j in range(16):
                        obuf[pl.ds(rh * 16 * D + j * D + d0, 16)] = v[j]
                return _2

            lax.fori_loop(0, RB // 16, rloop, None)
            pltpu.async_copy(obuf, out.at[pl.ds(r0 * D, RBLK)], sem)

        npair = ((nfull + NW - 1) // NW + 1) // 2

        @pl.when(wid < nfull)
        def _prime():
            fire_in(wid, in_buf, isem)

        def bloop(i, _, nfull=nfull, out=out):
            b0 = wid + (2 * i) * NW
            b1 = wid + (2 * i + 1) * NW
            b2 = wid + (2 * i + 2) * NW

            @pl.when(b1 < nfull)
            def _f1():
                fire_in(b1, in_buf2, isem2)

            @pl.when(b0 < nfull)
            def _p0():
                wait_in(b0, in_buf, isem)
                process(b0, in_buf, out_buf, osem)

            @pl.when(b2 < nfull)
            def _f2():
                fire_in(b2, in_buf, isem)

            @pl.when(b1 < nfull)
            def _p1():
                wait_in(b1, in_buf2, isem2)
                process(b1, in_buf2, out_buf2, osem2)
            return _

        lax.fori_loop(0, npair, bloop, None)

        nb = (nfull - wid + NW - 1) // NW

        @pl.when(nb >= 1)
        def _d0(out=out):
            pltpu.make_async_copy(out_buf, out.at[pl.ds(0, RBLK)],
                                  osem).wait()

        @pl.when(nb >= 2)
        def _d1(out=out):
            pltpu.make_async_copy(out_buf2, out.at[pl.ds(0, RBLK)],
                                  osem2).wait()


@functools.partial(
    pl.kernel,
    out_type=(jax.ShapeDtypeStruct((NU * D,), jnp.float32),
              jax.ShapeDtypeStruct((NI1 * D,), jnp.float32),
              jax.ShapeDtypeStruct((NI1 * D,), jnp.float32),
              jax.ShapeDtypeStruct((NI1 * D,), jnp.float32)),
    mesh=plsc.VectorSubcoreMesh(core_axis_name="c", subcore_axis_name="s"),
    scratch_types=[
        pltpu.VMEM((D, RB), jnp.float32),
        pltpu.VMEM((D, RB), jnp.float32),
        pltpu.VMEM((RBLK,), jnp.float32),
        pltpu.VMEM((RBLK,), jnp.float32),
        pltpu.SemaphoreType.DMA,
        pltpu.SemaphoreType.DMA,
        pltpu.SemaphoreType.DMA,
        pltpu.SemaphoreType.DMA,
    ],
    compiler_params=pltpu.CompilerParams(needs_layout_passes=False,
                                         use_tc_tiling_on_sc=True),
)
def _relayout_sc(euiT, eiuT, eilT, eliT, tEUI, tEIU, tEIL, tELI,
                 oEUI, oEIU, oEIL, oELI, *scratch):
    _relayout_body(euiT, eiuT, eilT, eliT, tEUI, tEIU, tEIL, tELI,
                   oEUI, oEIU, oEIL, oELI, *scratch)


def _sc_body(u_hbm, x_hbm, tar_hbm, neg_hbm, off_hbm, eui_t, eiu_t, eil_t,
             eli_t, outT_hbm, outN_hbm,
             u_v, x_v, tar_v, neg_v, off_v, eui_v,
             eli_rowsA, tarU_rowsA, tarL_rowsA, negU_rowsA, negL_rowsA,
             eli_rowsB, tarU_rowsB, tarL_rowsB, negU_rowsB, negL_rowsB,
             trT_v, trN_v, scT_v, scN_v, gsemA, gsemB, esem):
    wid = lax.axis_index("s") * NC + lax.axis_index("c")
    base = wid * BPT
    bufsA = (eli_rowsA, tarU_rowsA, tarL_rowsA, negU_rowsA, negL_rowsA)
    bufsB = (eli_rowsB, tarU_rowsB, tarL_rowsB, negU_rowsB, negL_rowsB)

    pltpu.sync_copy(u_hbm.at[pl.ds(base, BPT)], u_v)
    pltpu.sync_copy(x_hbm.at[pl.ds(wid * (BPT * L // XCOLS), BPT * L // XCOLS)], x_v)
    pltpu.sync_copy(tar_hbm.at[pl.ds(wid * NSTEP, NSTEP)], tar_v)
    pltpu.sync_copy(neg_hbm.at[pl.ds(wid * NSTEP, NSTEP)], neg_v)
    pltpu.sync_copy(off_hbm.at[pl.ds(base, BPT)], off_v)

    pltpu.async_copy(eui_t.at[u_v], eui_v, esem).wait()

    iot16 = lax.iota(jnp.int32, 16)

    def _copies(j, bufs, sem):
        eli_rows, tarU_rows, tarL_rows, negU_rows, negL_rows = bufs
        cps = []
        for g in range(XG):
            cps.append(pltpu.make_async_copy(
                eli_t.at[x_v.at[j * XG + g]],
                eli_rows.at[pl.ds(g * XCOLS, XCOLS)], sem))
        cps.append(pltpu.make_async_copy(eiu_t.at[tar_v.at[j]], tarU_rows, sem))
        cps.append(pltpu.make_async_copy(eil_t.at[tar_v.at[j]], tarL_rows, sem))
        cps.append(pltpu.make_async_copy(eiu_t.at[neg_v.at[j]], negU_rows, sem))
        cps.append(pltpu.make_async_copy(eil_t.at[neg_v.at[j]], negL_rows, sem))
        return cps

    def fire(j, bufs, sem):
        for c in _copies(j, bufs, sem):
            c.start()

    def drain(j, bufs, sem):
        for c in _copies(j, bufs, sem):
            c.wait()

    def compute(j, bufs):
        eli_rows, tarU_rows, tarL_rows, negU_rows, negL_rows = bufs
        for b in range(SB):
            bg = j * SB + b
            a = [eui_v[bg, pl.ds(16 * k, 16)] for k in range(4)]

            def lbody(l, e, b=b):
                r = b * L + l
                return tuple(e[k] + eli_rows[r, pl.ds(16 * k, 16)]
                             for k in range(4))

            e = lax.fori_loop(
                0, L, lbody,
                tuple(jnp.zeros((16,), jnp.float32) for _ in range(4)))
            off_s = plsc.load_gather(
                off_v, [jnp.full((16,), bg, jnp.int32)]) * (1.0 / L)
            e = [ek * off_s for ek in e]

            for t in range(T):
                r = b * T + t
                accT = a[0] * tarU_rows[r, pl.ds(0, 16)]
                accN = a[0] * negU_rows[r, pl.ds(0, 16)]
                for k in range(1, 4):
                    accT = accT + a[k] * tarU_rows[r, pl.ds(16 * k, 16)]
                    accN = accN + a[k] * negU_rows[r, pl.ds(16 * k, 16)]
                for k in range(4):
                    accT = accT + e[k] * tarL_rows[r, pl.ds(16 * k, 16)]
                    accN = accN + e[k] * negL_rows[r, pl.ds(16 * k, 16)]
                trT_v[pl.ds(r * 17, 16)] = accT
                trN_v[pl.ds(r * 17, 16)] = accN

        for g in range(NGRP):
            sT = jnp.zeros((16,), jnp.float32)
            sN = jnp.zeros((16,), jnp.float32)
            for c in range(16):
                idx = iot16 * 17 + (g * 272 + c)
                sT = sT + plsc.load_gather(trT_v, [idx])
                sN = sN + plsc.load_gather(trN_v, [idx])
            scT_v[pl.ds(j * PPS + g * 16, 16)] = sT
            scN_v[pl.ds(j * PPS + g * 16, 16)] = sN

    fire(0, bufsA, gsemA)

    def steppair(i, _):
        j0 = 2 * i
        j1 = 2 * i + 1
        fire(j1, bufsB, gsemB)
        drain(j0, bufsA, gsemA)
        compute(j0, bufsA)

        @pl.when(j1 + 1 < NSTEP)
        def _f():
            fire(j1 + 1, bufsA, gsemA)

        drain(j1, bufsB, gsemB)
        compute(j1, bufsB)
        return _

    lax.fori_loop(0, NSTEP // 2, steppair, None)

    pltpu.sync_copy(scT_v, outT_hbm.at[pl.ds(base * T, BPT * T)])
    pltpu.sync_copy(scN_v, outN_hbm.at[pl.ds(base * T, BPT * T)])


@functools.partial(
    pl.kernel,
    out_type=(jax.ShapeDtypeStruct((B * T,), jnp.float32),
              jax.ShapeDtypeStruct((B * T,), jnp.float32)),
    mesh=plsc.VectorSubcoreMesh(core_axis_name="c", subcore_axis_name="s"),
    scratch_types=[
        pltpu.VMEM((BPT,), jnp.int32),
        pltpu.VMEM((B * L // NW // XCOLS, XCOLS), jnp.int32),
        pltpu.VMEM((NSTEP, PPS), jnp.int32),
        pltpu.VMEM((NSTEP, PPS), jnp.int32),
        pltpu.VMEM((BPT,), jnp.float32),
        pltpu.VMEM((BPT, D), jnp.float32),
        pltpu.VMEM((SB * L, D), jnp.float32),
        pltpu.VMEM((PPS, D), jnp.float32),
        pltpu.VMEM((PPS, D), jnp.float32),
        pltpu.VMEM((PPS, D), jnp.float32),
        pltpu.VMEM((PPS, D), jnp.float32),
        pltpu.VMEM((SB * L, D), jnp.float32),
        pltpu.VMEM((PPS, D), jnp.float32),
        pltpu.VMEM((PPS, D), jnp.float32),
        pltpu.VMEM((PPS, D), jnp.float32),
        pltpu.VMEM((PPS, D), jnp.float32),
        pltpu.VMEM((PPS * 17,), jnp.float32),
        pltpu.VMEM((PPS * 17,), jnp.float32),
        pltpu.VMEM((BPT * T,), jnp.float32),
        pltpu.VMEM((BPT * T,), jnp.float32),
        pltpu.SemaphoreType.DMA,
        pltpu.SemaphoreType.DMA,
        pltpu.SemaphoreType.DMA,
    ],
    compiler_params=pltpu.CompilerParams(needs_layout_passes=False,
                                         use_tc_tiling_on_sc=False),
)
def _fpmc_sc(u, x2, tar2, neg2, off, EUI, EIU, EIL, ELI, outT, outN, *scratch):
    _sc_body(u, x2, tar2, neg2, off, EUI, EIU, EIL, ELI, outT, outN, *scratch)


def kernel(u, x, tar, neg, offset, isEval, EUI, EIU, EIL, ELI):
    nf = (NU // RB) * RB
    tEUI = EUI[nf:].reshape(-1)
    tEIU = EIU[nf:].reshape(-1)
    tEIL = EIL[nf:].reshape(-1)
    tELI = ELI[nf:].reshape(-1)
    eui_f, eiu_f, eil_f, eli_f = _relayout_sc(
        EUI.T, EIU.T, EIL.T, ELI.T, tEUI, tEIU, tEIL, tELI)
    EUIr = eui_f.reshape(NU, D)
    EIUr = eiu_f.reshape(NI1, D)
    EILr = eil_f.reshape(NI1, D)
    ELIr = eli_f.reshape(NI1, D)

    x2 = x.reshape(B * L // XCOLS, XCOLS)
    tar2 = tar.reshape(B * T // PPS, PPS)
    neg2 = neg.reshape(B * T // PPS, PPS)
    off = offset.reshape(B)
    sT, sN = _fpmc_sc(u, x2, tar2, neg2, off, EUIr, EIUr, EILr, ELIr)
    sT = sT.reshape(B, T)
    sN = sN.reshape(B, T)
    second = jnp.where(jnp.asarray(isEval), jnp.zeros_like(sN), sN)
    return (sT, second)


# speedup vs baseline: 1.0718x; 1.0093x over previous
import functools

import jax
import jax.numpy as jnp
from jax import lax
from jax.experimental import pallas as pl
from jax.experimental.pallas import tpu as pltpu
from jax.experimental.pallas import tpu_sc as plsc

B = 4096
L = 50
T = 10
D = 64
NU = 100000
NI1 = 100001

NC = 2
NS = 16
NW = NC * NS
BPT = B // NW
SB = 8
NSTEP = BPT // SB
XCOLS = 100
XG = SB * L // XCOLS
PPS = SB * T
NGRP = PPS // 16

RB = 128
RBLK = RB * D


def _relayout_body(euiT, eiuT, eilT, eliT, tEUI, tEIU, tEIL, tELI,
                   oEUI, oEIU, oEIL, oELI, in_buf, in_buf2, out_buf,
                   out_buf2, isem, isem2, osem, osem2):
    wid = lax.axis_index("s") * NC + lax.axis_index("c")
    iot = lax.iota(jnp.int32, 16)
    rotp = {s: (iot + s) & 15 for s in (8, 4, 2, 1)}
    rotm = {s: (iot - s) & 15 for s in (8, 4, 2, 1)}
    masks = {s: (iot & s) != 0 for s in (8, 4, 2, 1)}

    for ti, (tbl, tail, out, n_rows) in enumerate(
            ((euiT, tEUI, oEUI, NU), (eiuT, tEIU, oEIU, NI1),
             (eilT, tEIL, oEIL, NI1), (eliT, tELI, oELI, NI1))):
        nfull = n_rows // RB
        nt = n_rows - nfull * RB

        @pl.when(wid == NW - 1 - ti)
        def _tail(tail=tail, out=out, nfull=nfull, nt=nt):
            pltpu.sync_copy(tail, out_buf.at[pl.ds(0, nt * D)])
            pltpu.sync_copy(out_buf.at[pl.ds(0, nt * D)],
                            out.at[pl.ds(nfull * RB * D, nt * D)])

        def fire_in(bid, buf, sem, tbl=tbl):
            r0 = pl.multiple_of(bid * RB, RB)
            for dhi in range(D // 8):
                pltpu.async_copy(tbl.at[pl.ds(dhi * 8, 8), pl.ds(r0, RB)],
                                 buf.at[pl.ds(dhi * 8, 8)], sem)

        def wait_in(bid, buf, sem, tbl=tbl):
            r0 = pl.multiple_of(bid * RB, RB)
            pltpu.make_async_copy(tbl.at[:, pl.ds(r0, RB)], buf, sem).wait()

        def process(bid, buf, obuf, sem, out=out):
            r0 = pl.multiple_of(bid * RB, RB)

            @pl.when(bid >= 2 * NW)
            def _drain():
                pltpu.make_async_copy(
                    obuf, out.at[pl.ds(r0 * D, RBLK)], sem).wait()

            def rloop(rh, _2):
                for d0 in range(0, D, 16):
                    v = [buf[d0 + i, pl.ds(rh * 16, 16)] for i in range(16)]
                    for s in (8, 4, 2, 1):
                        m = masks[s]
                        for i in range(16):
                            if i & s:
                                continue
                            a, b = v[i], v[i | s]
                            ra = a.at[rotp[s]].get(mode="promise_in_bounds")
                            rb = b.at[rotm[s]].get(mode="promise_in_bounds")
                            v[i] = jnp.where(m, rb, a)
                            v[i | s] = jnp.where(m, b, ra)
                    for j in range(16):
                        obuf[pl.ds(rh * 16 * D + j * D + d0, 16)] = v[j]
                return _2

            lax.fori_loop(0, RB // 16, rloop, None)
            pltpu.async_copy(obuf, out.at[pl.ds(r0 * D, RBLK)], sem)

        npair = ((nfull + NW - 1) // NW + 1) // 2

        @pl.when(wid < nfull)
        def _prime():
            fire_in(wid, in_buf, isem)

        def bloop(i, _, nfull=nfull, out=out):
            b0 = wid + (2 * i) * NW
            b1 = wid + (2 * i + 1) * NW
            b2 = wid + (2 * i + 2) * NW

            @pl.when(b1 < nfull)
            def _f1():
                fire_in(b1, in_buf2, isem2)

            @pl.when(b0 < nfull)
            def _p0():
                wait_in(b0, in_buf, isem)
                process(b0, in_buf, out_buf, osem)

            @pl.when(b2 < nfull)
            def _f2():
                fire_in(b2, in_buf, isem)

            @pl.when(b1 < nfull)
            def _p1():
                wait_in(b1, in_buf2, isem2)
                process(b1, in_buf2, out_buf2, osem2)
            return _

        lax.fori_loop(0, npair, bloop, None)

        nb = (nfull - wid + NW - 1) // NW

        @pl.when(nb >= 1)
        def _d0(out=out):
            pltpu.make_async_copy(out_buf, out.at[pl.ds(0, RBLK)],
                                  osem).wait()

        @pl.when(nb >= 2)
        def _d1(out=out):
            pltpu.make_async_copy(out_buf2, out.at[pl.ds(0, RBLK)],
                                  osem2).wait()


@functools.partial(
    pl.kernel,
    out_type=(jax.ShapeDtypeStruct((NU * D,), jnp.float32),
              jax.ShapeDtypeStruct((NI1 * D,), jnp.float32),
              jax.ShapeDtypeStruct((NI1 * D,), jnp.float32),
              jax.ShapeDtypeStruct((NI1 * D,), jnp.float32)),
    mesh=plsc.VectorSubcoreMesh(core_axis_name="c", subcore_axis_name="s"),
    scratch_types=[
        pltpu.VMEM((D, RB), jnp.float32),
        pltpu.VMEM((D, RB), jnp.float32),
        pltpu.VMEM((RBLK,), jnp.float32),
        pltpu.VMEM((RBLK,), jnp.float32),
        pltpu.SemaphoreType.DMA,
        pltpu.SemaphoreType.DMA,
        pltpu.SemaphoreType.DMA,
        pltpu.SemaphoreType.DMA,
    ],
    compiler_params=pltpu.CompilerParams(needs_layout_passes=False,
                                         use_tc_tiling_on_sc=True),
)
def _relayout_sc(euiT, eiuT, eilT, eliT, tEUI, tEIU, tEIL, tELI,
                 oEUI, oEIU, oEIL, oELI, *scratch):
    _relayout_body(euiT, eiuT, eilT, eliT, tEUI, tEIU, tEIL, tELI,
                   oEUI, oEIU, oEIL, oELI, *scratch)


def _sc_body(u_hbm, x_hbm, tar_hbm, neg_hbm, off_hbm, eui_t, eiu_t, eil_t,
             eli_t, outT_hbm, outN_hbm,
             u_v, x_v, tar_v, neg_v, off_v, eui_v,
             eli_rowsA, tarU_rowsA, tarL_rowsA, negU_rowsA, negL_rowsA,
             eli_rowsB, tarU_rowsB, tarL_rowsB, negU_rowsB, negL_rowsB,
             trT_v, trN_v, scT_v, scN_v, gsemA, gsemB, esem):
    wid = lax.axis_index("s") * NC + lax.axis_index("c")
    base = wid * BPT
    bufsA = (eli_rowsA, tarU_rowsA, tarL_rowsA, negU_rowsA, negL_rowsA)
    bufsB = (eli_rowsB, tarU_rowsB, tarL_rowsB, negU_rowsB, negL_rowsB)

    pltpu.sync_copy(u_hbm.at[pl.ds(base, BPT)], u_v)
    pltpu.sync_copy(x_hbm.at[pl.ds(wid * (BPT * L // XCOLS), BPT * L // XCOLS)], x_v)
    pltpu.sync_copy(tar_hbm.at[pl.ds(wid * NSTEP, NSTEP)], tar_v)
    pltpu.sync_copy(neg_hbm.at[pl.ds(wid * NSTEP, NSTEP)], neg_v)
    pltpu.sync_copy(off_hbm.at[pl.ds(base, BPT)], off_v)

    pltpu.async_copy(eui_t.at[u_v], eui_v, esem).wait()

    iot16 = lax.iota(jnp.int32, 16)

    def _copies(j, bufs, sem):
        eli_rows, tarU_rows, tarL_rows, negU_rows, negL_rows = bufs
        cps = []
        for g in range(XG):
            cps.append(pltpu.make_async_copy(
                eli_t.at[x_v.at[j * XG + g]],
                eli_rows.at[pl.ds(g * XCOLS, XCOLS)], sem))
        cps.append(pltpu.make_async_copy(eiu_t.at[tar_v.at[j]], tarU_rows, sem))
        cps.append(pltpu.make_async_copy(eil_t.at[tar_v.at[j]], tarL_rows, sem))
        cps.append(pltpu.make_async_copy(eiu_t.at[neg_v.at[j]], negU_rows, sem))
        cps.append(pltpu.make_async_copy(eil_t.at[neg_v.at[j]], negL_rows, sem))
        return cps

    def fire(j, bufs, sem):
        for c in _copies(j, bufs, sem):
            c.start()

    def drain(j, bufs, sem):
        for c in _copies(j, bufs, sem):
            c.wait()

    def compute(j, bufs):
        eli_rows, tarU_rows, tarL_rows, negU_rows, negL_rows = bufs
        for b in range(SB):
            bg = j * SB + b
            a = [eui_v[bg, pl.ds(16 * k, 16)] for k in range(4)]

            def lbody(l, e8, b=b):
                r = b * L + 2 * l
                return (tuple(e8[k] + eli_rows[r, pl.ds(16 * k, 16)]
                              for k in range(4)) +
                        tuple(e8[4 + k] + eli_rows[r + 1, pl.ds(16 * k, 16)]
                              for k in range(4)))

            e8 = lax.fori_loop(
                0, L // 2, lbody,
                tuple(jnp.zeros((16,), jnp.float32) for _ in range(8)))
            e = [e8[k] + e8[4 + k] for k in range(4)]
            off_s = plsc.load_gather(
                off_v, [jnp.full((16,), bg, jnp.int32)]) * (1.0 / L)
            e = [ek * off_s for ek in e]

            for t in range(T):
                r = b * T + t
                accT = a[0] * tarU_rows[r, pl.ds(0, 16)]
                accN = a[0] * negU_rows[r, pl.ds(0, 16)]
                for k in range(1, 4):
                    accT = accT + a[k] * tarU_rows[r, pl.ds(16 * k, 16)]
                    accN = accN + a[k] * negU_rows[r, pl.ds(16 * k, 16)]
                for k in range(4):
                    accT = accT + e[k] * tarL_rows[r, pl.ds(16 * k, 16)]
                    accN = accN + e[k] * negL_rows[r, pl.ds(16 * k, 16)]
                trT_v[pl.ds(r * 17, 16)] = accT
                trN_v[pl.ds(r * 17, 16)] = accN

        for g in range(NGRP):
            sT = jnp.zeros((16,), jnp.float32)
            sN = jnp.zeros((16,), jnp.float32)
            for c in range(16):
                idx = iot16 * 17 + (g * 272 + c)
                sT = sT + plsc.load_gather(trT_v, [idx])
                sN = sN + plsc.load_gather(trN_v, [idx])
            scT_v[pl.ds(j * PPS + g * 16, 16)] = sT
            scN_v[pl.ds(j * PPS + g * 16, 16)] = sN

    fire(0, bufsA, gsemA)

    def steppair(i, _):
        j0 = 2 * i
        j1 = 2 * i + 1
        fire(j1, bufsB, gsemB)
        drain(j0, bufsA, gsemA)
        compute(j0, bufsA)

        @pl.when(j1 + 1 < NSTEP)
        def _f():
            fire(j1 + 1, bufsA, gsemA)

        drain(j1, bufsB, gsemB)
        compute(j1, bufsB)
        return _

    lax.fori_loop(0, NSTEP // 2, steppair, None)

    pltpu.sync_copy(scT_v, outT_hbm.at[pl.ds(base * T, BPT * T)])
    pltpu.sync_copy(scN_v, outN_hbm.at[pl.ds(base * T, BPT * T)])


@functools.partial(
    pl.kernel,
    out_type=(jax.ShapeDtypeStruct((B * T,), jnp.float32),
              jax.ShapeDtypeStruct((B * T,), jnp.float32)),
    mesh=plsc.VectorSubcoreMesh(core_axis_name="c", subcore_axis_name="s"),
    scratch_types=[
        pltpu.VMEM((BPT,), jnp.int32),
        pltpu.VMEM((B * L // NW // XCOLS, XCOLS), jnp.int32),
        pltpu.VMEM((NSTEP, PPS), jnp.int32),
        pltpu.VMEM((NSTEP, PPS), jnp.int32),
        pltpu.VMEM((BPT,), jnp.float32),
        pltpu.VMEM((BPT, D), jnp.float32),
        pltpu.VMEM((SB * L, D), jnp.float32),
        pltpu.VMEM((PPS, D), jnp.float32),
        pltpu.VMEM((PPS, D), jnp.float32),
        pltpu.VMEM((PPS, D), jnp.float32),
        pltpu.VMEM((PPS, D), jnp.float32),
        pltpu.VMEM((SB * L, D), jnp.float32),
        pltpu.VMEM((PPS, D), jnp.float32),
        pltpu.VMEM((PPS, D), jnp.float32),
        pltpu.VMEM((PPS, D), jnp.float32),
        pltpu.VMEM((PPS, D), jnp.float32),
        pltpu.VMEM((PPS * 17,), jnp.float32),
        pltpu.VMEM((PPS * 17,), jnp.float32),
        pltpu.VMEM((BPT * T,), jnp.float32),
        pltpu.VMEM((BPT * T,), jnp.float32),
        pltpu.SemaphoreType.DMA,
        pltpu.SemaphoreType.DMA,
        pltpu.SemaphoreType.DMA,
    ],
    compiler_params=pltpu.CompilerParams(needs_layout_passes=False,
                                         use_tc_tiling_on_sc=False),
)
def _fpmc_sc(u, x2, tar2, neg2, off, EUI, EIU, EIL, ELI, outT, outN, *scratch):
    _sc_body(u, x2, tar2, neg2, off, EUI, EIU, EIL, ELI, outT, outN, *scratch)


def kernel(u, x, tar, neg, offset, isEval, EUI, EIU, EIL, ELI):
    nf = (NU // RB) * RB
    tEUI = EUI[nf:].reshape(-1)
    tEIU = EIU[nf:].reshape(-1)
    tEIL = EIL[nf:].reshape(-1)
    tELI = ELI[nf:].reshape(-1)
    eui_f, eiu_f, eil_f, eli_f = _relayout_sc(
        EUI.T, EIU.T, EIL.T, ELI.T, tEUI, tEIU, tEIL, tELI)
    EUIr = eui_f.reshape(NU, D)
    EIUr = eiu_f.reshape(NI1, D)
    EILr = eil_f.reshape(NI1, D)
    ELIr = eli_f.reshape(NI1, D)

    x2 = x.reshape(B * L // XCOLS, XCOLS)
    tar2 = tar.reshape(B * T // PPS, PPS)
    neg2 = neg.reshape(B * T // PPS, PPS)
    off = offset.reshape(B)
    sT, sN = _fpmc_sc(u, x2, tar2, neg2, off, EUIr, EIUr, EILr, ELIr)
    sT = sT.reshape(B, T)
    sN = sN.reshape(B, T)
    second = jnp.where(jnp.asarray(isEval), jnp.zeros_like(sN), sN)
    return (sT, second)
